# baseline (speedup 1.0000x reference)
import jax
import jax.numpy as jnp
from jax.experimental import pallas as pl
from jax.experimental.pallas import tpu as pltpu


def _body(mid_ref, x_ref, pe_ref, table_ref, out_ref):
    mid = mid_ref[0]
    row = table_ref[pl.ds(mid, 1), :]
    out_ref[...] = x_ref[...] + pe_ref[...] + row


@jax.jit
def kernel(x, temporal_pe, modality_table, modality_id):
    B, T, D = x.shape
    TB = 2048
    nt = T // TB
    x2 = x.reshape(B * T, D)
    pe2 = temporal_pe.reshape(temporal_pe.shape[1], D)
    mid = jnp.asarray(modality_id, jnp.int32).reshape(1)

    grid_spec = pltpu.PrefetchScalarGridSpec(
        num_scalar_prefetch=1,
        grid=(nt, B),
        in_specs=[
            pl.BlockSpec((TB, D), lambda t, b, mid: (b * nt + t, 0)),
            pl.BlockSpec((TB, D), lambda t, b, mid: (t, 0)),
            pl.BlockSpec(modality_table.shape, lambda t, b, mid: (0, 0)),
        ],
        out_specs=pl.BlockSpec((TB, D), lambda t, b, mid: (b * nt + t, 0)),
    )

    out = pl.pallas_call(
        _body,
        grid_spec=grid_spec,
        out_shape=jax.ShapeDtypeStruct((B * T, D), x.dtype),
        compiler_params=pltpu.CompilerParams(
            dimension_semantics=("parallel", "parallel"),
        ),
    )(mid, x2, pe2, modality_table)
    return out.reshape(B, T, D)
